# Initial kernel scaffold; baseline (speedup 1.0000x reference)
import jax
import jax.numpy as jnp
from jax.experimental import pallas as pl


def kernel(x, Wr, W1, b1, W2, b2):
    raise NotImplementedError("write your pallas kernel here")


# speedup vs baseline: 1.0276x; 1.0276x over previous
import functools

import jax
import jax.numpy as jnp
from jax import lax
from jax.experimental import pallas as pl
from jax.experimental.pallas import tpu as pltpu

N = 2048
D = 768
F = 3072
E = 8
TS = 128
NT = 24
N_PAD = NT * TS
X_ROWS = N + 8


def _router_body(x_ref, wr_ref, idx_ref, score_ref):
    l = jnp.dot(x_ref[...], wr_ref[...], preferred_element_type=jnp.float32)
    m = jnp.max(l, axis=1, keepdims=True)
    denom = jnp.sum(jnp.exp(l - m), axis=1, keepdims=True)
    lanes = lax.broadcasted_iota(jnp.int32, l.shape, 1)
    idx = jnp.min(jnp.where(l == m, lanes, E), axis=1, keepdims=True)
    idx_ref[...] = idx
    score_ref[...] = 1.0 / denom


def _router(x_flat, Wr):
    return pl.pallas_call(
        _router_body,
        out_shape=(
            jax.ShapeDtypeStruct((N, 1), jnp.int32),
            jax.ShapeDtypeStruct((N, 1), jnp.float32),
        ),
    )(x_flat, Wr)


def _ffn_body(te_ref, tv_ref, x_ref, w1_ref, b1_ref, w2_ref, b2_ref, s_ref,
              o_ref):
    t = pl.program_id(0)

    @pl.when(tv_ref[t] != 0)
    def _():
        xt = x_ref[...]
        h = jnp.dot(xt, w1_ref[0], preferred_element_type=jnp.float32)
        h = jnp.maximum(h + b1_ref[0], 0.0)
        o = jnp.dot(h, w2_ref[0], preferred_element_type=jnp.float32)
        o_ref[...] = (o + b2_ref[0]) * s_ref[...]


def _ffn(tile_expert, tile_valid, x_sorted, W1, b1, W2, b2, scores_sorted):
    grid_spec = pltpu.PrefetchScalarGridSpec(
        num_scalar_prefetch=2,
        grid=(NT,),
        in_specs=[
            pl.BlockSpec((TS, D), lambda t, te, tv: (t, 0)),
            pl.BlockSpec((1, D, F), lambda t, te, tv: (te[t], 0, 0)),
            pl.BlockSpec((1, 1, F), lambda t, te, tv: (te[t], 0, 0)),
            pl.BlockSpec((1, F, D), lambda t, te, tv: (te[t], 0, 0)),
            pl.BlockSpec((1, 1, D), lambda t, te, tv: (te[t], 0, 0)),
            pl.BlockSpec((TS, 1), lambda t, te, tv: (t, 0)),
        ],
        out_specs=pl.BlockSpec((TS, D), lambda t, te, tv: (t, 0)),
    )
    return pl.pallas_call(
        _ffn_body,
        grid_spec=grid_spec,
        out_shape=jax.ShapeDtypeStruct((N_PAD, D), jnp.float32),
    )(tile_expert, tile_valid, x_sorted, W1, b1.reshape(E, 1, F), W2,
      b2.reshape(E, 1, D), scores_sorted)


def kernel(x, Wr, W1, b1, W2, b2):
    B, T, Dm = x.shape
    x_flat = x.reshape(N, D)

    expert_idx, score = _router(x_flat, Wr)
    expert_idx = expert_idx[:, 0]
    score = score[:, 0]

    counts = jnp.sum(expert_idx[None, :] == jnp.arange(E)[:, None], axis=1)
    c_pad = ((counts + TS - 1) // TS) * TS
    starts_pad = jnp.concatenate(
        [jnp.zeros((1,), jnp.int32), jnp.cumsum(c_pad)[:-1]]).astype(jnp.int32)
    starts_compact = jnp.concatenate(
        [jnp.zeros((1,), jnp.int32), jnp.cumsum(counts)[:-1]]).astype(jnp.int32)
    order = jnp.argsort(expert_idx, stable=True)
    e_sorted = expert_idx[order]
    rank = jnp.arange(N, dtype=jnp.int32) - starts_compact[e_sorted]
    pos_sorted = starts_pad[e_sorted] + rank
    sort_idx = jnp.full((N_PAD,), N, jnp.int32).at[pos_sorted].set(
        order.astype(jnp.int32))
    total_tiles = jnp.sum(c_pad) // TS
    tile_ids = jnp.arange(NT, dtype=jnp.int32)
    tile_expert = jnp.sum(
        tile_ids[:, None] * TS >= starts_pad[None, 1:], axis=1).astype(jnp.int32)
    tile_valid = (tile_ids < total_tiles).astype(jnp.int32)

    x_pad = jnp.concatenate(
        [x_flat, jnp.zeros((X_ROWS - N, D), jnp.float32)], axis=0)
    x_sorted = x_pad[sort_idx]
    scores_sorted = jnp.where(sort_idx < N, score[jnp.minimum(sort_idx, N - 1)],
                              0.0)[:, None]

    o_sorted = _ffn(tile_expert, tile_valid, x_sorted, W1, b1, W2, b2,
                    scores_sorted)

    inv_pos = jnp.zeros((N,), jnp.int32).at[order].set(
        pos_sorted.astype(jnp.int32))
    out = o_sorted[inv_pos]
    return out.reshape(B, T, Dm)


# speedup vs baseline: 1.2678x; 1.2338x over previous
import functools

import jax
import jax.numpy as jnp
from jax import lax
from jax.experimental import pallas as pl
from jax.experimental.pallas import tpu as pltpu
from jax.experimental.pallas import tpu_sc as plsc

N = 2048
D = 768
F = 3072
E = 8
TS = 128
NT = 24
N_PAD = NT * TS
NTOK = N // TS


def _router_body(x_ref, wr_ref, idx_ref, score_ref):
    l = jnp.dot(x_ref[...], wr_ref[...], preferred_element_type=jnp.float32)
    m = jnp.max(l, axis=1, keepdims=True)
    denom = jnp.sum(jnp.exp(l - m), axis=1, keepdims=True)
    lanes = lax.broadcasted_iota(jnp.int32, l.shape, 1)
    idx = jnp.min(jnp.where(l == m, lanes, E), axis=1, keepdims=True)
    idx_ref[...] = idx
    score_ref[...] = 1.0 / denom


def _router(x_flat, Wr):
    return pl.pallas_call(
        _router_body,
        out_shape=(
            jax.ShapeDtypeStruct((N, 1), jnp.int32),
            jax.ShapeDtypeStruct((N, 1), jnp.float32),
        ),
    )(x_flat, Wr)


def _ffn_body(te_ref, tv_ref, si_ref, x_ref, w1_ref, b1_ref, w2_ref, b2_ref,
              o_ref):
    t = pl.program_id(0)

    @pl.when(tv_ref[t] != 0)
    def _():
        toks = lax.broadcasted_iota(jnp.int32, (TS, N), 1)
        p = (si_ref[...] == toks).astype(jnp.float32)
        xt = jnp.dot(p, x_ref[...], preferred_element_type=jnp.float32)
        h = jnp.dot(xt, w1_ref[0], preferred_element_type=jnp.float32)
        h = jnp.maximum(h + b1_ref[0], 0.0)
        o = jnp.dot(h, w2_ref[0], preferred_element_type=jnp.float32)
        o_ref[...] = o + b2_ref[0]


def _ffn(tile_expert, tile_valid, sort_idx_col, x_flat, W1, b1, W2, b2):
    grid_spec = pltpu.PrefetchScalarGridSpec(
        num_scalar_prefetch=2,
        grid=(NT,),
        in_specs=[
            pl.BlockSpec((TS, 1), lambda t, te, tv: (t, 0)),
            pl.BlockSpec((N, D), lambda t, te, tv: (0, 0)),
            pl.BlockSpec((1, D, F), lambda t, te, tv: (te[t], 0, 0)),
            pl.BlockSpec((1, 1, F), lambda t, te, tv: (te[t], 0, 0)),
            pl.BlockSpec((1, F, D), lambda t, te, tv: (te[t], 0, 0)),
            pl.BlockSpec((1, 1, D), lambda t, te, tv: (te[t], 0, 0)),
        ],
        out_specs=pl.BlockSpec((TS, D), lambda t, te, tv: (t, 0)),
    )
    return pl.pallas_call(
        _ffn_body,
        grid_spec=grid_spec,
        out_shape=jax.ShapeDtypeStruct((N_PAD, D), jnp.float32),
    )(tile_expert, tile_valid, sort_idx_col, x_flat, W1,
      b1.reshape(E, 1, F), W2, b2.reshape(E, 1, D))


def _unsort_body(si_ref, o_ref, s_ref, out_ref):
    tau = pl.program_id(0)
    toks = lax.broadcasted_iota(jnp.int32, (TS, N_PAD), 0) + tau * TS
    g = (toks == si_ref[...]).astype(jnp.float32)
    out = jnp.dot(g, o_ref[...], preferred_element_type=jnp.float32)
    out_ref[...] = out * s_ref[...]


def _unsort(sort_idx_row, o_sorted, score):
    return pl.pallas_call(
        _unsort_body,
        grid=(NTOK,),
        in_specs=[
            pl.BlockSpec((1, N_PAD), lambda t: (0, 0)),
            pl.BlockSpec((N_PAD, D), lambda t: (0, 0)),
            pl.BlockSpec((TS, 1), lambda t: (t, 0)),
        ],
        out_specs=pl.BlockSpec((TS, D), lambda t: (t, 0)),
        out_shape=jax.ShapeDtypeStruct((N, D), jnp.float32),
    )(sort_idx_row, o_sorted, score)


def kernel(x, Wr, W1, b1, W2, b2):
    B, T, Dm = x.shape
    x_flat = x.reshape(N, D)

    expert_idx, score = _router(x_flat, Wr)
    expert_idx = expert_idx[:, 0]

    counts = jnp.sum(expert_idx[None, :] == jnp.arange(E)[:, None], axis=1)
    c_pad = ((counts + TS - 1) // TS) * TS
    starts_pad = jnp.concatenate(
        [jnp.zeros((1,), jnp.int32), jnp.cumsum(c_pad)[:-1]]).astype(jnp.int32)
    starts_compact = jnp.concatenate(
        [jnp.zeros((1,), jnp.int32), jnp.cumsum(counts)[:-1]]).astype(jnp.int32)
    order = jnp.argsort(expert_idx, stable=True)
    e_sorted = expert_idx[order]
    rank = jnp.arange(N, dtype=jnp.int32) - starts_compact[e_sorted]
    pos_sorted = starts_pad[e_sorted] + rank
    sort_idx = jnp.full((N_PAD,), N, jnp.int32).at[pos_sorted].set(
        order.astype(jnp.int32))
    total_tiles = jnp.sum(c_pad) // TS
    tile_ids = jnp.arange(NT, dtype=jnp.int32)
    tile_expert = jnp.sum(
        tile_ids[:, None] * TS >= starts_pad[None, 1:], axis=1).astype(jnp.int32)
    tile_valid = (tile_ids < total_tiles).astype(jnp.int32)

    o_sorted = _ffn(tile_expert, tile_valid, sort_idx[:, None], x_flat,
                    W1, b1, W2, b2)
    out = _unsort(sort_idx[None, :], o_sorted, score)
    return out.reshape(B, T, Dm)


# speedup vs baseline: 1.5121x; 1.1927x over previous
import functools

import jax
import jax.numpy as jnp
from jax import lax
from jax.experimental import pallas as pl
from jax.experimental.pallas import tpu as pltpu
from jax.experimental.pallas import tpu_sc as plsc

N = 2048
D = 768
F = 3072
E = 8
TS = 128
NT = 24
N_PAD = NT * TS
NTOK = N // TS


NW = 32
_SC_MESH = dict(core_axis_name="c", subcore_axis_name="s")
NCHUNK = N // 16


def _sc_sort(expert_idx):
    @functools.partial(
        pl.kernel,
        mesh=plsc.VectorSubcoreMesh(**_SC_MESH),
        compiler_params=pltpu.CompilerParams(needs_layout_passes=False),
        out_type=(
            jax.ShapeDtypeStruct((N_PAD,), jnp.int32),
            jax.ShapeDtypeStruct((NW,), jnp.int32),
            jax.ShapeDtypeStruct((NW,), jnp.int32),
        ),
        scratch_types=[
            pltpu.VMEM((N,), jnp.int32),
            pltpu.VMEM((N + 2 * TS,), jnp.int32),
            pltpu.VMEM((TS,), jnp.int32),
            pltpu.VMEM((NW,), jnp.int32),
            pltpu.VMEM((NW,), jnp.int32),
        ],
    )
    def k(eidx_hbm, sidx_hbm, te_hbm, tv_hbm, ids_v, comp_v, sent_v, te_v,
          tv_v):
        wid = lax.axis_index("s") * 2 + lax.axis_index("c")

        @pl.when(wid < E)
        def _():
            e = wid
            pltpu.sync_copy(eidx_hbm, ids_v)
            lane = lax.iota(jnp.int32, 16)
            zeros = jnp.zeros((16,), jnp.int32)

            ev = zeros + e
            trash = N + 2 * TS - 16

            def body(c, carry):
                off = carry[0]
                toks = carry[1]
                h = carry[2:]
                v = ids_v[pl.ds(c * 16, 16)]
                mi = 1 - jnp.minimum(jnp.abs(v - ev), 1)
                pos_real = off + plsc.cumsum(mi) - 1
                pos = pos_real * mi + trash * (1 - mi)
                plsc.store_scatter(comp_v, [pos], toks)
                newh = tuple(
                    h[j] + 1 - jnp.minimum(jnp.abs(v - j), 1)
                    for j in range(E))
                return (off + jnp.sum(mi), toks + 16) + newh

            init = (jnp.int32(0), lane) + tuple(zeros for _ in range(E))
            res = lax.fori_loop(0, NCHUNK, body, init)
            c_own = res[0]
            counts = [jnp.sum(res[2 + j]) for j in range(E)]
            c_pads = [((c + TS - 1) // TS) * TS for c in counts]
            start = jnp.int32(0)
            for j in range(E):
                start = start + jnp.where(j < e, c_pads[j], 0)

            sent = jnp.full((16,), N, jnp.int32)
            for j in range(TS // 16):
                comp_v[pl.ds(c_own + j * 16, 16)] = sent

            c_pad_own = ((c_own + TS - 1) // TS) * TS

            def dma_body(kk, _):
                pltpu.sync_copy(
                    comp_v.at[pl.ds(kk * TS, TS)],
                    sidx_hbm.at[pl.ds(pl.multiple_of(start + kk * TS, TS),
                                      TS)])
                return 0

            lax.fori_loop(0, c_pad_own // TS, dma_body, 0)

            total_pad = jnp.int32(0)
            for j in range(E):
                total_pad = total_pad + c_pads[j]

            @pl.when(e == E - 1)
            def _():
                for j in range(TS // 16):
                    sent_v[pl.ds(j * 16, 16)] = sent

                def fill_body(kk, _):
                    pltpu.sync_copy(
                        sent_v,
                        sidx_hbm.at[pl.ds(pl.multiple_of(kk * TS, TS), TS)])
                    return 0

                lax.fori_loop(total_pad // TS, NT, fill_body, 0)

            @pl.when(e == 0)
            def _():
                total_tiles = total_pad // TS
                for t in range(NW // 16):
                    tid = lane + t * 16
                    te = jnp.zeros((16,), jnp.int32)
                    stile = jnp.int32(0)
                    for j in range(1, E):
                        stile = stile + c_pads[j - 1] // TS
                        te = te + jnp.minimum(
                            jnp.maximum(tid - stile + 1, 0), 1)
                    te_v[pl.ds(t * 16, 16)] = te
                    tv_v[pl.ds(t * 16, 16)] = jnp.minimum(
                        jnp.maximum(total_tiles - tid, 0), 1)
                pltpu.sync_copy(te_v, te_hbm)
                pltpu.sync_copy(tv_v, tv_hbm)

    return k(expert_idx)


def _router_body(x_ref, wr_ref, idx_ref, score_ref):
    l = jnp.dot(x_ref[...], wr_ref[...], preferred_element_type=jnp.float32)
    m = jnp.max(l, axis=1, keepdims=True)
    denom = jnp.sum(jnp.exp(l - m), axis=1, keepdims=True)
    lanes = lax.broadcasted_iota(jnp.int32, l.shape, 1)
    idx = jnp.min(jnp.where(l == m, lanes, E), axis=1, keepdims=True)
    idx_ref[...] = idx
    score_ref[...] = 1.0 / denom


def _router(x_flat, Wr):
    return pl.pallas_call(
        _router_body,
        out_shape=(
            jax.ShapeDtypeStruct((N, 1), jnp.int32),
            jax.ShapeDtypeStruct((N, 1), jnp.float32),
        ),
    )(x_flat, Wr)


def _ffn_body(te_ref, tv_ref, si_ref, x_ref, w1_ref, b1_ref, w2_ref, b2_ref,
              o_ref):
    t = pl.program_id(0)

    @pl.when(tv_ref[t] != 0)
    def _():
        toks = lax.broadcasted_iota(jnp.int32, (TS, N), 1)
        p = (si_ref[...] == toks).astype(jnp.float32)
        xt = jnp.dot(p, x_ref[...], preferred_element_type=jnp.float32)
        h = jnp.dot(xt, w1_ref[0], preferred_element_type=jnp.float32)
        h = jnp.maximum(h + b1_ref[0], 0.0)
        o = jnp.dot(h, w2_ref[0], preferred_element_type=jnp.float32)
        o_ref[...] = o + b2_ref[0]


def _ffn(tile_expert, tile_valid, sort_idx_col, x_flat, W1, b1, W2, b2):
    grid_spec = pltpu.PrefetchScalarGridSpec(
        num_scalar_prefetch=2,
        grid=(NT,),
        in_specs=[
            pl.BlockSpec((TS, 1), lambda t, te, tv: (t, 0)),
            pl.BlockSpec((N, D), lambda t, te, tv: (0, 0)),
            pl.BlockSpec((1, D, F), lambda t, te, tv: (te[t], 0, 0)),
            pl.BlockSpec((1, 1, F), lambda t, te, tv: (te[t], 0, 0)),
            pl.BlockSpec((1, F, D), lambda t, te, tv: (te[t], 0, 0)),
            pl.BlockSpec((1, 1, D), lambda t, te, tv: (te[t], 0, 0)),
        ],
        out_specs=pl.BlockSpec((TS, D), lambda t, te, tv: (t, 0)),
    )
    return pl.pallas_call(
        _ffn_body,
        grid_spec=grid_spec,
        out_shape=jax.ShapeDtypeStruct((N_PAD, D), jnp.float32),
    )(tile_expert, tile_valid, sort_idx_col, x_flat, W1,
      b1.reshape(E, 1, F), W2, b2.reshape(E, 1, D))


def _unsort_body(si_ref, o_ref, s_ref, out_ref):
    tau = pl.program_id(0)
    toks = lax.broadcasted_iota(jnp.int32, (TS, N_PAD), 0) + tau * TS
    g = (toks == si_ref[...]).astype(jnp.float32)
    out = jnp.dot(g, o_ref[...], preferred_element_type=jnp.float32)
    out_ref[...] = out * s_ref[...]


def _unsort(sort_idx_row, o_sorted, score):
    return pl.pallas_call(
        _unsort_body,
        grid=(NTOK,),
        in_specs=[
            pl.BlockSpec((1, N_PAD), lambda t: (0, 0)),
            pl.BlockSpec((N_PAD, D), lambda t: (0, 0)),
            pl.BlockSpec((TS, 1), lambda t: (t, 0)),
        ],
        out_specs=pl.BlockSpec((TS, D), lambda t: (t, 0)),
        out_shape=jax.ShapeDtypeStruct((N, D), jnp.float32),
    )(sort_idx_row, o_sorted, score)


def kernel(x, Wr, W1, b1, W2, b2):
    B, T, Dm = x.shape
    x_flat = x.reshape(N, D)

    expert_idx, score = _router(x_flat, Wr)

    sort_idx, tile_expert, tile_valid = _sc_sort(expert_idx[:, 0])

    o_sorted = _ffn(tile_expert, tile_valid, sort_idx[:, None], x_flat,
                    W1, b1, W2, b2)
    out = _unsort(sort_idx[None, :], o_sorted, score)
    return out.reshape(B, T, Dm)


# speedup vs baseline: 1.8325x; 1.2119x over previous
import functools

import jax
import jax.numpy as jnp
from jax import lax
from jax.experimental import pallas as pl
from jax.experimental.pallas import tpu as pltpu
from jax.experimental.pallas import tpu_sc as plsc

N = 2048
D = 768
F = 3072
E = 8
TS = 128
NT = 24
N_PAD = NT * TS
NTOK = N // TS


NW = 32
_SC_MESH = dict(core_axis_name="c", subcore_axis_name="s")
NCHUNK = N // 16


def _sc_sort(expert_idx):
    @functools.partial(
        pl.kernel,
        mesh=plsc.VectorSubcoreMesh(**_SC_MESH),
        compiler_params=pltpu.CompilerParams(needs_layout_passes=False),
        out_type=(
            jax.ShapeDtypeStruct((N_PAD,), jnp.int32),
            jax.ShapeDtypeStruct((16,), jnp.int32),
        ),
        scratch_types=[
            pltpu.VMEM((N,), jnp.int32),
            pltpu.VMEM((N + 2 * TS,), jnp.int32),
            pltpu.VMEM((TS,), jnp.int32),
            pltpu.VMEM((16,), jnp.int32),
        ],
    )
    def k(eidx_hbm, sidx_hbm, st_hbm, ids_v, comp_v, sent_v, st_v):
        wid = lax.axis_index("s") * 2 + lax.axis_index("c")

        @pl.when(wid < E)
        def _():
            e = wid
            pltpu.sync_copy(eidx_hbm, ids_v)
            lane = lax.iota(jnp.int32, 16)
            zeros = jnp.zeros((16,), jnp.int32)

            ev = zeros + e
            trash = N + 2 * TS - 16

            def body(c, carry):
                off = carry[0]
                toks = carry[1]
                h = carry[2:]
                v = ids_v[pl.ds(c * 16, 16)]
                mi = 1 - jnp.minimum(jnp.abs(v - ev), 1)
                pos_real = off + plsc.cumsum(mi) - 1
                pos = pos_real * mi + trash * (1 - mi)
                plsc.store_scatter(comp_v, [pos], toks)
                newh = tuple(
                    h[j] + 1 - jnp.minimum(jnp.abs(v - j), 1)
                    for j in range(E))
                return (off + jnp.sum(mi), toks + 16) + newh

            init = (jnp.int32(0), lane) + tuple(zeros for _ in range(E))
            res = lax.fori_loop(0, NCHUNK, body, init)
            c_own = res[0]
            counts = [jnp.sum(res[2 + j]) for j in range(E)]
            c_pads = [((c + TS - 1) // TS) * TS for c in counts]
            start = jnp.int32(0)
            for j in range(E):
                start = start + jnp.where(j < e, c_pads[j], 0)

            sent = jnp.full((16,), N, jnp.int32)
            for j in range(TS // 16):
                comp_v[pl.ds(c_own + j * 16, 16)] = sent

            c_pad_own = ((c_own + TS - 1) // TS) * TS

            def dma_body(kk, _):
                pltpu.sync_copy(
                    comp_v.at[pl.ds(kk * TS, TS)],
                    sidx_hbm.at[pl.ds(pl.multiple_of(start + kk * TS, TS),
                                      TS)])
                return 0

            lax.fori_loop(0, c_pad_own // TS, dma_body, 0)

            total_pad = jnp.int32(0)
            for j in range(E):
                total_pad = total_pad + c_pads[j]

            @pl.when(e == E - 1)
            def _():
                for j in range(TS // 16):
                    sent_v[pl.ds(j * 16, 16)] = sent

                def fill_body(kk, _):
                    pltpu.sync_copy(
                        sent_v,
                        sidx_hbm.at[pl.ds(pl.multiple_of(kk * TS, TS), TS)])
                    return 0

                lax.fori_loop(total_pad // TS, NT, fill_body, 0)

            @pl.when(e == 0)
            def _():
                st = jnp.zeros((16,), jnp.int32)
                for j in range(E):
                    st = st + (c_pads[j] // TS) * jnp.minimum(
                        jnp.maximum(lane - j, 0), 1)
                st_v[...] = st
                pltpu.sync_copy(st_v, st_hbm)

    return k(expert_idx)


def _router_body(x_ref, wr_ref, idx_ref, score_ref):
    l = jnp.dot(x_ref[...], wr_ref[...], preferred_element_type=jnp.float32)
    m = jnp.max(l, axis=1, keepdims=True)
    denom = jnp.sum(jnp.exp(l - m), axis=1, keepdims=True)
    lanes = lax.broadcasted_iota(jnp.int32, l.shape, 1)
    idx = jnp.min(jnp.where(l == m, lanes, E), axis=1, keepdims=True)
    idx_ref[...] = idx
    score_ref[...] = 1.0 / denom


def _router(x_flat, Wr):
    return pl.pallas_call(
        _router_body,
        out_shape=(
            jax.ShapeDtypeStruct((N, 1), jnp.int32),
            jax.ShapeDtypeStruct((N, 1), jnp.float32),
        ),
    )(x_flat, Wr)


def _ffn_body(st_ref, si_ref, x_ref, w1_hbm, b1_ref, w2_hbm, b2_ref, o_ref,
              w1b, w2b, wsem):
    o_ref[...] = jnp.zeros((N_PAD, D), jnp.float32)

    def wcopy(e, slot):
        return (
            pltpu.make_async_copy(w1_hbm.at[e], w1b.at[slot],
                                  wsem.at[slot, 0]),
            pltpu.make_async_copy(w2_hbm.at[e], w2b.at[slot],
                                  wsem.at[slot, 1]),
        )

    for c in wcopy(0, 0):
        c.start()
    for e in range(E):
        slot = e % 2
        if e + 1 < E:
            for c in wcopy(e + 1, 1 - slot):
                c.start()
        for c in wcopy(e, slot):
            c.wait()

        def tile_body(t, _):
            row = pl.multiple_of(t * TS, TS)
            si = si_ref[pl.ds(row, TS), :]
            toks = lax.broadcasted_iota(jnp.int32, (TS, N), 1)
            p = (si == toks).astype(jnp.float32)
            xt = jnp.dot(p, x_ref[...], preferred_element_type=jnp.float32)
            h = jnp.dot(xt, w1b[slot], preferred_element_type=jnp.float32)
            h = jnp.maximum(h + b1_ref[pl.ds(e, 1)], 0.0)
            o = jnp.dot(h, w2b[slot], preferred_element_type=jnp.float32)
            o_ref[pl.ds(row, TS), :] = o + b2_ref[pl.ds(e, 1)]
            return 0

        lax.fori_loop(st_ref[e], st_ref[e + 1], tile_body, 0)


def _ffn(start_tile, sort_idx_col, x_flat, W1, b1, W2, b2):
    return pl.pallas_call(
        _ffn_body,
        in_specs=[
            pl.BlockSpec(memory_space=pltpu.SMEM),
            pl.BlockSpec(memory_space=pltpu.VMEM),
            pl.BlockSpec(memory_space=pltpu.VMEM),
            pl.BlockSpec(memory_space=pl.ANY),
            pl.BlockSpec(memory_space=pltpu.VMEM),
            pl.BlockSpec(memory_space=pl.ANY),
            pl.BlockSpec(memory_space=pltpu.VMEM),
        ],
        scratch_shapes=[
            pltpu.VMEM((2, D, F), jnp.float32),
            pltpu.VMEM((2, F, D), jnp.float32),
            pltpu.SemaphoreType.DMA((2, 2)),
        ],
        out_shape=jax.ShapeDtypeStruct((N_PAD, D), jnp.float32),
    )(start_tile, sort_idx_col, x_flat, W1, b1, W2, b2)


def _unsort_body(si_ref, o_ref, s_ref, out_ref):
    tau = pl.program_id(0)
    toks = lax.broadcasted_iota(jnp.int32, (TS, N_PAD), 0) + tau * TS
    g = (toks == si_ref[...]).astype(jnp.float32)
    out = jnp.dot(g, o_ref[...], preferred_element_type=jnp.float32)
    out_ref[...] = out * s_ref[...]


def _unsort(sort_idx_row, o_sorted, score):
    return pl.pallas_call(
        _unsort_body,
        grid=(NTOK,),
        in_specs=[
            pl.BlockSpec((1, N_PAD), lambda t: (0, 0)),
            pl.BlockSpec((N_PAD, D), lambda t: (0, 0)),
            pl.BlockSpec((TS, 1), lambda t: (t, 0)),
        ],
        out_specs=pl.BlockSpec((TS, D), lambda t: (t, 0)),
        out_shape=jax.ShapeDtypeStruct((N, D), jnp.float32),
    )(sort_idx_row, o_sorted, score)


def kernel(x, Wr, W1, b1, W2, b2):
    B, T, Dm = x.shape
    x_flat = x.reshape(N, D)

    expert_idx, score = _router(x_flat, Wr)

    sort_idx, start_tile = _sc_sort(expert_idx[:, 0])

    o_sorted = _ffn(start_tile, sort_idx[:, None], x_flat, W1, b1, W2, b2)
    out = _unsort(sort_idx[None, :], o_sorted, score)
    return out.reshape(B, T, Dm)


# speedup vs baseline: 1.8733x; 1.0223x over previous
import functools

import jax
import jax.numpy as jnp
from jax import lax
from jax.experimental import pallas as pl
from jax.experimental.pallas import tpu as pltpu
from jax.experimental.pallas import tpu_sc as plsc

N = 2048
D = 768
F = 3072
E = 8
TS = 128
NT = 24
N_PAD = NT * TS
NTOK = N // TS


NW = 32
_SC_MESH = dict(core_axis_name="c", subcore_axis_name="s")
NCHUNK = N // 16


def _sc_sort(expert_idx):
    @functools.partial(
        pl.kernel,
        mesh=plsc.VectorSubcoreMesh(**_SC_MESH),
        compiler_params=pltpu.CompilerParams(needs_layout_passes=False),
        out_type=(
            jax.ShapeDtypeStruct((N_PAD,), jnp.int32),
            jax.ShapeDtypeStruct((16,), jnp.int32),
        ),
        scratch_types=[
            pltpu.VMEM((N,), jnp.int32),
            pltpu.VMEM((N + 2 * TS,), jnp.int32),
            pltpu.VMEM((TS,), jnp.int32),
            pltpu.VMEM((16,), jnp.int32),
        ],
    )
    def k(eidx_hbm, sidx_hbm, st_hbm, ids_v, comp_v, sent_v, st_v):
        wid = lax.axis_index("s") * 2 + lax.axis_index("c")

        @pl.when(wid < E)
        def _():
            e = wid
            pltpu.sync_copy(eidx_hbm, ids_v)
            lane = lax.iota(jnp.int32, 16)
            zeros = jnp.zeros((16,), jnp.int32)

            ev = zeros + e
            trash = N + 2 * TS - 16

            def body(c, carry):
                off = carry[0]
                toks = carry[1]
                h = carry[2:]
                v = ids_v[pl.ds(c * 16, 16)]
                mi = 1 - jnp.minimum(jnp.abs(v - ev), 1)
                pos_real = off + plsc.cumsum(mi) - 1
                pos = pos_real * mi + trash * (1 - mi)
                plsc.store_scatter(comp_v, [pos], toks)
                newh = tuple(
                    h[j] + 1 - jnp.minimum(jnp.abs(v - j), 1)
                    for j in range(E))
                return (off + jnp.sum(mi), toks + 16) + newh

            init = (jnp.int32(0), lane) + tuple(zeros for _ in range(E))
            res = lax.fori_loop(0, NCHUNK, body, init)
            c_own = res[0]
            counts = [jnp.sum(res[2 + j]) for j in range(E)]
            c_pads = [((c + TS - 1) // TS) * TS for c in counts]
            start = jnp.int32(0)
            for j in range(E):
                start = start + jnp.where(j < e, c_pads[j], 0)

            sent = jnp.full((16,), N, jnp.int32)
            for j in range(TS // 16):
                comp_v[pl.ds(c_own + j * 16, 16)] = sent

            c_pad_own = ((c_own + TS - 1) // TS) * TS

            def dma_body(kk, _):
                pltpu.sync_copy(
                    comp_v.at[pl.ds(kk * TS, TS)],
                    sidx_hbm.at[pl.ds(pl.multiple_of(start + kk * TS, TS),
                                      TS)])
                return 0

            lax.fori_loop(0, c_pad_own // TS, dma_body, 0)

            total_pad = jnp.int32(0)
            for j in range(E):
                total_pad = total_pad + c_pads[j]

            @pl.when(e == E - 1)
            def _():
                for j in range(TS // 16):
                    sent_v[pl.ds(j * 16, 16)] = sent

                def fill_body(kk, _):
                    pltpu.sync_copy(
                        sent_v,
                        sidx_hbm.at[pl.ds(pl.multiple_of(kk * TS, TS), TS)])
                    return 0

                lax.fori_loop(total_pad // TS, NT, fill_body, 0)

            @pl.when(e == 0)
            def _():
                st = jnp.zeros((16,), jnp.int32)
                for j in range(E):
                    st = st + (c_pads[j] // TS) * jnp.minimum(
                        jnp.maximum(lane - j, 0), 1)
                st_v[...] = st
                pltpu.sync_copy(st_v, st_hbm)

    return k(expert_idx)


def _router_body(x_ref, wr_ref, idx_ref, score_ref):
    l = jnp.dot(x_ref[...], wr_ref[...], preferred_element_type=jnp.float32)
    m = jnp.max(l, axis=1, keepdims=True)
    denom = jnp.sum(jnp.exp(l - m), axis=1, keepdims=True)
    lanes = lax.broadcasted_iota(jnp.int32, l.shape, 1)
    idx = jnp.min(jnp.where(l == m, lanes, E), axis=1)
    idx_ref[...] = idx
    score_ref[...] = 1.0 / denom


def _router(x_flat, Wr):
    return pl.pallas_call(
        _router_body,
        out_shape=(
            jax.ShapeDtypeStruct((N,), jnp.int32),
            jax.ShapeDtypeStruct((N, 1), jnp.float32),
        ),
    )(x_flat, Wr)


def _ffn_body(st_ref, si_ref, x_ref, w1_hbm, b1_ref, w2_hbm, b2_ref, o_ref,
              w1b, w2b, wsem):
    o_ref[...] = jnp.zeros((N_PAD, D), jnp.float32)

    def wcopy(e, slot):
        return (
            pltpu.make_async_copy(w1_hbm.at[e], w1b.at[slot],
                                  wsem.at[slot, 0]),
            pltpu.make_async_copy(w2_hbm.at[e], w2b.at[slot],
                                  wsem.at[slot, 1]),
        )

    for c in wcopy(0, 0):
        c.start()
    for e in range(E):
        slot = e % 2
        if e + 1 < E:
            for c in wcopy(e + 1, 1 - slot):
                c.start()
        for c in wcopy(e, slot):
            c.wait()

        def tile_body(t, _):
            row = pl.multiple_of(t * TS, TS)
            si = si_ref[pl.ds(row, TS), :]
            toks = lax.broadcasted_iota(jnp.int32, (TS, N), 1)
            p = (si == toks).astype(jnp.float32)
            xt = jnp.dot(p, x_ref[...], preferred_element_type=jnp.float32)
            h = jnp.dot(xt, w1b[slot], preferred_element_type=jnp.float32)
            h = jnp.maximum(h + b1_ref[pl.ds(e, 1)], 0.0)
            o = jnp.dot(h, w2b[slot], preferred_element_type=jnp.float32)
            o_ref[pl.ds(row, TS), :] = o + b2_ref[pl.ds(e, 1)]
            return 0

        lax.fori_loop(st_ref[e], st_ref[e + 1], tile_body, 0)


def _ffn(start_tile, sort_idx_col, x_flat, W1, b1, W2, b2):
    return pl.pallas_call(
        _ffn_body,
        in_specs=[
            pl.BlockSpec(memory_space=pltpu.SMEM),
            pl.BlockSpec(memory_space=pltpu.VMEM),
            pl.BlockSpec(memory_space=pltpu.VMEM),
            pl.BlockSpec(memory_space=pl.ANY),
            pl.BlockSpec(memory_space=pltpu.VMEM),
            pl.BlockSpec(memory_space=pl.ANY),
            pl.BlockSpec(memory_space=pltpu.VMEM),
        ],
        scratch_shapes=[
            pltpu.VMEM((2, D, F), jnp.float32),
            pltpu.VMEM((2, F, D), jnp.float32),
            pltpu.SemaphoreType.DMA((2, 2)),
        ],
        out_shape=jax.ShapeDtypeStruct((N_PAD, D), jnp.float32),
    )(start_tile, sort_idx_col, x_flat, W1, b1, W2, b2)


def _unsort_body(si_ref, o_ref, s_ref, out_ref):
    tau = pl.program_id(0)
    toks = lax.broadcasted_iota(jnp.int32, (TS, N_PAD), 0) + tau * TS
    g = (toks == si_ref[...]).astype(jnp.float32)
    out = jnp.dot(g, o_ref[...], preferred_element_type=jnp.float32)
    out_ref[...] = out * s_ref[...]


def _unsort(sort_idx_row, o_sorted, score):
    return pl.pallas_call(
        _unsort_body,
        grid=(NTOK,),
        in_specs=[
            pl.BlockSpec((1, N_PAD), lambda t: (0, 0)),
            pl.BlockSpec((N_PAD, D), lambda t: (0, 0)),
            pl.BlockSpec((TS, 1), lambda t: (t, 0)),
        ],
        out_specs=pl.BlockSpec((TS, D), lambda t: (t, 0)),
        out_shape=jax.ShapeDtypeStruct((N, D), jnp.float32),
    )(sort_idx_row, o_sorted, score)


def kernel(x, Wr, W1, b1, W2, b2):
    B, T, Dm = x.shape
    x_flat = x.reshape(N, D)

    expert_idx, score = _router(x_flat, Wr)

    sort_idx, start_tile = _sc_sort(expert_idx)

    o_sorted = _ffn(start_tile, sort_idx[:, None], x_flat, W1, b1, W2, b2)
    out = _unsort(sort_idx[None, :], o_sorted, score)
    return out.reshape(B, T, Dm)


# speedup vs baseline: 1.8801x; 1.0036x over previous
import functools

import jax
import jax.numpy as jnp
from jax import lax
from jax.experimental import pallas as pl
from jax.experimental.pallas import tpu as pltpu
from jax.experimental.pallas import tpu_sc as plsc

N = 2048
D = 768
F = 3072
E = 8
TS = 128
NT = 24
N_PAD = NT * TS
NTOK = N // TS


NW = 32
_SC_MESH = dict(core_axis_name="c", subcore_axis_name="s")
NCHUNK = N // 16


def _sc_sort(expert_idx):
    @functools.partial(
        pl.kernel,
        mesh=plsc.VectorSubcoreMesh(**_SC_MESH),
        compiler_params=pltpu.CompilerParams(needs_layout_passes=False),
        out_type=(
            jax.ShapeDtypeStruct((N_PAD,), jnp.int32),
            jax.ShapeDtypeStruct((16,), jnp.int32),
        ),
        scratch_types=[
            pltpu.VMEM((N,), jnp.int32),
            pltpu.VMEM((N + 2 * TS,), jnp.int32),
            pltpu.VMEM((TS,), jnp.int32),
            pltpu.VMEM((16,), jnp.int32),
        ],
    )
    def k(eidx_hbm, sidx_hbm, st_hbm, ids_v, comp_v, sent_v, st_v):
        wid = lax.axis_index("s") * 2 + lax.axis_index("c")

        @pl.when(wid < E)
        def _():
            e = wid
            pltpu.sync_copy(eidx_hbm, ids_v)
            lane = lax.iota(jnp.int32, 16)
            zeros = jnp.zeros((16,), jnp.int32)

            ev = zeros + e
            trash = N + 2 * TS - 16

            def body(c, carry):
                off = carry[0]
                toks = carry[1]
                h = carry[2:]
                v = ids_v[pl.ds(c * 16, 16)]
                mi = 1 - jnp.minimum(jnp.abs(v - ev), 1)
                pos_real = off + plsc.cumsum(mi) - 1
                pos = pos_real * mi + trash * (1 - mi)
                plsc.store_scatter(comp_v, [pos], toks)
                newh = tuple(
                    h[j] + 1 - jnp.minimum(jnp.abs(v - j), 1)
                    for j in range(E))
                return (off + jnp.sum(mi), toks + 16) + newh

            init = (jnp.int32(0), lane) + tuple(zeros for _ in range(E))
            res = lax.fori_loop(0, NCHUNK, body, init)
            c_own = res[0]
            counts = [jnp.sum(res[2 + j]) for j in range(E)]
            c_pads = [((c + TS - 1) // TS) * TS for c in counts]
            start = jnp.int32(0)
            for j in range(E):
                start = start + jnp.where(j < e, c_pads[j], 0)

            sent = jnp.full((16,), N, jnp.int32)
            for j in range(TS // 16):
                comp_v[pl.ds(c_own + j * 16, 16)] = sent

            c_pad_own = ((c_own + TS - 1) // TS) * TS

            def dma_body(kk, _):
                pltpu.sync_copy(
                    comp_v.at[pl.ds(kk * TS, TS)],
                    sidx_hbm.at[pl.ds(pl.multiple_of(start + kk * TS, TS),
                                      TS)])
                return 0

            lax.fori_loop(0, c_pad_own // TS, dma_body, 0)

            total_pad = jnp.int32(0)
            for j in range(E):
                total_pad = total_pad + c_pads[j]

            @pl.when(e == E - 1)
            def _():
                for j in range(TS // 16):
                    sent_v[pl.ds(j * 16, 16)] = sent

                def fill_body(kk, _):
                    pltpu.sync_copy(
                        sent_v,
                        sidx_hbm.at[pl.ds(pl.multiple_of(kk * TS, TS), TS)])
                    return 0

                lax.fori_loop(total_pad // TS, NT, fill_body, 0)

            @pl.when(e == 0)
            def _():
                st = jnp.zeros((16,), jnp.int32)
                for j in range(E):
                    st = st + (c_pads[j] // TS) * jnp.minimum(
                        jnp.maximum(lane - j, 0), 1)
                st_v[...] = st
                pltpu.sync_copy(st_v, st_hbm)

    return k(expert_idx)


def _router_body(x_ref, wr_ref, idx_ref, score_ref):
    l = jnp.dot(x_ref[...], wr_ref[...], preferred_element_type=jnp.float32)
    m = jnp.max(l, axis=1, keepdims=True)
    denom = jnp.sum(jnp.exp(l - m), axis=1, keepdims=True)
    lanes = lax.broadcasted_iota(jnp.int32, l.shape, 1)
    idx = jnp.min(jnp.where(l == m, lanes, E), axis=1)
    idx_ref[...] = idx
    score_ref[...] = 1.0 / denom


def _router(x_flat, Wr):
    return pl.pallas_call(
        _router_body,
        out_shape=(
            jax.ShapeDtypeStruct((N,), jnp.int32),
            jax.ShapeDtypeStruct((N, 1), jnp.float32),
        ),
    )(x_flat, Wr)


def _ffn_body(st_ref, si_ref, x_ref, w1_hbm, b1_ref, w2_hbm, b2_ref, o_ref,
              w1b, w2b, wsem):
    def wcopy(e, slot):
        return (
            pltpu.make_async_copy(w1_hbm.at[e], w1b.at[slot],
                                  wsem.at[slot, 0]),
            pltpu.make_async_copy(w2_hbm.at[e], w2b.at[slot],
                                  wsem.at[slot, 1]),
        )

    for c in wcopy(0, 0):
        c.start()
    o_ref[...] = jnp.zeros((N_PAD, D), jnp.float32)
    for e in range(E):
        slot = e % 2
        if e + 1 < E:
            for c in wcopy(e + 1, 1 - slot):
                c.start()
        for c in wcopy(e, slot):
            c.wait()

        def tile_body(t, _):
            row = pl.multiple_of(t * TS, TS)
            si = si_ref[pl.ds(row, TS), :]
            toks = lax.broadcasted_iota(jnp.int32, (TS, N), 1)
            p = (si == toks).astype(jnp.float32)
            xt = jnp.dot(p, x_ref[...], preferred_element_type=jnp.float32)
            h = jnp.dot(xt, w1b[slot], preferred_element_type=jnp.float32)
            h = jnp.maximum(h + b1_ref[pl.ds(e, 1)], 0.0)
            o = jnp.dot(h, w2b[slot], preferred_element_type=jnp.float32)
            o_ref[pl.ds(row, TS), :] = o + b2_ref[pl.ds(e, 1)]
            return 0

        lax.fori_loop(st_ref[e], st_ref[e + 1], tile_body, 0)


def _ffn(start_tile, sort_idx_col, x_flat, W1, b1, W2, b2):
    return pl.pallas_call(
        _ffn_body,
        in_specs=[
            pl.BlockSpec(memory_space=pltpu.SMEM),
            pl.BlockSpec(memory_space=pltpu.VMEM),
            pl.BlockSpec(memory_space=pltpu.VMEM),
            pl.BlockSpec(memory_space=pl.ANY),
            pl.BlockSpec(memory_space=pltpu.VMEM),
            pl.BlockSpec(memory_space=pl.ANY),
            pl.BlockSpec(memory_space=pltpu.VMEM),
        ],
        scratch_shapes=[
            pltpu.VMEM((2, D, F), jnp.float32),
            pltpu.VMEM((2, F, D), jnp.float32),
            pltpu.SemaphoreType.DMA((2, 2)),
        ],
        out_shape=jax.ShapeDtypeStruct((N_PAD, D), jnp.float32),
    )(start_tile, sort_idx_col, x_flat, W1, b1, W2, b2)


def _unsort_body(si_ref, o_ref, s_ref, out_ref):
    tau = pl.program_id(0)
    toks = lax.broadcasted_iota(jnp.int32, (TS, N_PAD), 0) + tau * TS
    g = (toks == si_ref[...]).astype(jnp.float32)
    out = jnp.dot(g, o_ref[...], preferred_element_type=jnp.float32)
    out_ref[...] = out * s_ref[...]


def _unsort(sort_idx_row, o_sorted, score):
    return pl.pallas_call(
        _unsort_body,
        grid=(NTOK,),
        in_specs=[
            pl.BlockSpec((1, N_PAD), lambda t: (0, 0)),
            pl.BlockSpec((N_PAD, D), lambda t: (0, 0)),
            pl.BlockSpec((TS, 1), lambda t: (t, 0)),
        ],
        out_specs=pl.BlockSpec((TS, D), lambda t: (t, 0)),
        out_shape=jax.ShapeDtypeStruct((N, D), jnp.float32),
    )(sort_idx_row, o_sorted, score)


def kernel(x, Wr, W1, b1, W2, b2):
    B, T, Dm = x.shape
    x_flat = x.reshape(N, D)

    expert_idx, score = _router(x_flat, Wr)

    sort_idx, start_tile = _sc_sort(expert_idx)

    o_sorted = _ffn(start_tile, sort_idx[:, None], x_flat, W1, b1, W2, b2)
    out = _unsort(sort_idx[None, :], o_sorted, score)
    return out.reshape(B, T, Dm)
